# baseline (speedup 1.0000x reference)
import functools

import jax
import jax.numpy as jnp
from jax import lax
from jax.experimental import pallas as pl
from jax.experimental.pallas import tpu as pltpu
from jax.experimental.pallas import tpu_sc as plsc

_EPS = 0.1
_NC = 100000
_INV_EPS = _EPS / (_NC - 1)
_B = 1024
_COEF = (1.0 - _EPS) - _INV_EPS

_ROWS = 8192
_COLS = 12500
_BLK_R = 128


def _sum_body(x_ref, o_ref):
    @pl.when(pl.program_id(0) == 0)
    def _init():
        o_ref[0, 0] = 0.0

    o_ref[0, 0] += jnp.sum(x_ref[...])


def _dense_sum(x2d):
    return pl.pallas_call(
        _sum_body,
        grid=(_ROWS // _BLK_R,),
        in_specs=[pl.BlockSpec((_BLK_R, _COLS), lambda i: (i, 0))],
        out_specs=pl.BlockSpec((1, 1), lambda i: (0, 0), memory_space=pltpu.SMEM),
        out_shape=jax.ShapeDtypeStruct((1, 1), jnp.float32),
    )(x2d)


_info = plsc.get_sparse_core_info()
_NCORES = _info.num_cores
_NSUB = _info.num_subcores
_NW = _NCORES * _NSUB
_RPW = _B // _NW
_L = 16


def _sc_gather_body(pred_hbm, tgt_hbm, out_hbm, tgt_v, idx_v, vals_v, part_v, sem):
    wid = lax.axis_index("s") * _NCORES + lax.axis_index("c")
    base = wid * _RPW
    pltpu.sync_copy(tgt_hbm.at[pl.ds(base, _RPW)], tgt_v)
    for c in range(_RPW // _L):
        t = tgt_v[pl.ds(c * _L, _L)]
        rows = lax.iota(jnp.int32, _L) + (base + c * _L)
        idx_v[pl.ds(c * _L, _L)] = rows * _NC + t
    pltpu.async_copy(pred_hbm.at[idx_v], vals_v, sem).wait()
    part_v[...] = vals_v[pl.ds(0, _L)] + vals_v[pl.ds(_L, _L)]
    pltpu.sync_copy(part_v, out_hbm.at[wid])


_sc_gather = functools.partial(
    pl.kernel,
    mesh=plsc.VectorSubcoreMesh(core_axis_name="c", subcore_axis_name="s"),
    out_type=jax.ShapeDtypeStruct((_NW, _L), jnp.float32),
    scratch_types=[
        pltpu.VMEM((_RPW,), jnp.int32),
        pltpu.VMEM((_RPW,), jnp.int32),
        pltpu.VMEM((_RPW,), jnp.float32),
        pltpu.VMEM((_L,), jnp.float32),
        pltpu.SemaphoreType.DMA,
    ],
)(_sc_gather_body)


def kernel(predictions, targets):
    total = _dense_sum(predictions.reshape(_ROWS, _COLS))[0, 0]
    parts = _sc_gather(predictions.reshape(-1), targets)
    return -(_INV_EPS * total + _COEF * jnp.sum(parts)) / _B


# speedup vs baseline: 1.2648x; 1.2648x over previous
import functools

import jax
import jax.numpy as jnp
from jax import lax
from jax.experimental import pallas as pl
from jax.experimental.pallas import tpu as pltpu
from jax.experimental.pallas import tpu_sc as plsc

_EPS = 0.1
_NC = 100000
_INV_EPS = _EPS / (_NC - 1)
_B = 1024
_COEF = (1.0 - _EPS) - _INV_EPS

_BLK_R = 32


def _sum_body(x_ref, o_ref):
    @pl.when(pl.program_id(0) == 0)
    def _init():
        o_ref[0, 0] = 0.0

    o_ref[0, 0] += jnp.sum(x_ref[...])


def _dense_sum(x2d):
    return pl.pallas_call(
        _sum_body,
        grid=(_B // _BLK_R,),
        in_specs=[pl.BlockSpec((_BLK_R, _NC), lambda i: (i, 0))],
        out_specs=pl.BlockSpec((1, 1), lambda i: (0, 0), memory_space=pltpu.SMEM),
        out_shape=jax.ShapeDtypeStruct((1, 1), jnp.float32),
    )(x2d)


_info = plsc.get_sparse_core_info()
_NCORES = _info.num_cores
_NSUB = _info.num_subcores
_NW = _NCORES * _NSUB
_RPW = _B // _NW
_L = 16


def _sc_gather_body(pred_hbm, tgt_hbm, out_hbm, tgt_v, idx_v, vals_v, part_v, sem):
    wid = lax.axis_index("s") * _NCORES + lax.axis_index("c")
    base = wid * _RPW
    pltpu.sync_copy(tgt_hbm.at[pl.ds(base, _RPW)], tgt_v)
    for c in range(_RPW // _L):
        t = tgt_v[pl.ds(c * _L, _L)]
        rows = lax.iota(jnp.int32, _L) + (base + c * _L)
        idx_v[pl.ds(c * _L, _L)] = rows * _NC + t
    pltpu.async_copy(pred_hbm.at[idx_v], vals_v, sem).wait()
    part_v[...] = vals_v[pl.ds(0, _L)] + vals_v[pl.ds(_L, _L)]
    pltpu.sync_copy(part_v, out_hbm.at[wid])


_sc_gather = functools.partial(
    pl.kernel,
    mesh=plsc.VectorSubcoreMesh(core_axis_name="c", subcore_axis_name="s"),
    out_type=jax.ShapeDtypeStruct((_NW, _L), jnp.float32),
    scratch_types=[
        pltpu.VMEM((_RPW,), jnp.int32),
        pltpu.VMEM((_RPW,), jnp.int32),
        pltpu.VMEM((_RPW,), jnp.float32),
        pltpu.VMEM((_L,), jnp.float32),
        pltpu.SemaphoreType.DMA,
    ],
)(_sc_gather_body)


def kernel(predictions, targets):
    total = _dense_sum(predictions)[0, 0]
    parts = _sc_gather(predictions.reshape(-1), targets)
    return -(_INV_EPS * total + _COEF * jnp.sum(parts)) / _B


# speedup vs baseline: 2.6712x; 2.1119x over previous
import functools

import jax
import jax.numpy as jnp
from jax import lax
from jax.experimental import pallas as pl
from jax.experimental.pallas import tpu as pltpu
from jax.experimental.pallas import tpu_sc as plsc

_EPS = 0.1
_NC = 100000
_INV_EPS = _EPS / (_NC - 1)
_B = 1024
_COEF = (1.0 - _EPS) - _INV_EPS

_BLK_R = 32


_TAIL = 32
_TAIL_START = _NC - _TAIL


def _sum_body(x_ref, tgt_ref, o_ref):
    @pl.when(pl.program_id(0) == 0)
    def _init():
        o_ref[0, 0] = 0.0
        o_ref[0, 1] = 0.0

    o_ref[0, 0] += jnp.sum(x_ref[...])
    xtail = x_ref[:, _TAIL_START:]
    col = jax.lax.broadcasted_iota(jnp.int32, (_BLK_R, _TAIL), 1) + _TAIL_START
    hit = col == tgt_ref[...]
    o_ref[0, 1] += jnp.sum(jnp.where(hit, xtail, 0.0))


def _dense_sum(x2d, tgt2d):
    return pl.pallas_call(
        _sum_body,
        grid=(_B // _BLK_R,),
        in_specs=[
            pl.BlockSpec((_BLK_R, _NC), lambda i: (i, 0)),
            pl.BlockSpec((_BLK_R, 1), lambda i: (i, 0)),
        ],
        out_specs=pl.BlockSpec((1, 2), lambda i: (0, 0), memory_space=pltpu.SMEM),
        out_shape=jax.ShapeDtypeStruct((1, 2), jnp.float32),
    )(x2d, tgt2d)


_info = plsc.get_sparse_core_info()
_NCORES = _info.num_cores
_NSUB = _info.num_subcores
_NW = _NCORES * _NSUB
_RPW = _B // _NW
_L = 16


_TW = 128
_TH = 8
_MAXC0 = _NC - _TAIL - _TW


def _sc_gather_body(pred_hbm, tgt_hbm, out_hbm, tgt_v, win_v, part_v, sem):
    wid = lax.axis_index("s") * _NCORES + lax.axis_index("c")
    base = wid * _RPW
    pltpu.sync_copy(tgt_hbm.at[pl.ds(base, _RPW)], tgt_v)
    iota = lax.iota(jnp.int32, _L)
    copies = []
    scalars = []
    tvecs = [tgt_v[pl.ds(c * _L, _L)] for c in range(_RPW // _L)]
    for r in range(_RPW):
        t = tvecs[r // _L][r % _L]
        c0 = pl.multiple_of(
            jnp.minimum(jnp.bitwise_and(t, -_TW), _MAXC0), _TW
        )
        tile_row = pl.multiple_of(base + (r // _TH) * _TH, _TH)
        copies.append(
            pltpu.async_copy(
                pred_hbm.at[pl.ds(tile_row, _TH), pl.ds(c0, _TW)],
                win_v.at[r],
                sem,
            )
        )
        scalars.append((t, c0))
    for cp in copies:
        cp.wait()
    acc = jnp.zeros((_L,), jnp.float32)
    for r in range(_RPW):
        t, c0 = scalars[r]
        lane = jnp.minimum(t - c0, _TW - 1)
        chunk = jnp.bitwise_and(lane, -_L)
        v16 = win_v[r, r % _TH, pl.ds(chunk, _L)]
        hit = jnp.where(iota == lane - chunk, v16, 0.0)
        acc = acc + hit * jnp.where(t < _TAIL_START, 1.0, 0.0)
    part_v[...] = acc
    pltpu.sync_copy(part_v, out_hbm.at[wid])


_sc_gather = functools.partial(
    pl.kernel,
    mesh=plsc.VectorSubcoreMesh(core_axis_name="c", subcore_axis_name="s"),
    out_type=jax.ShapeDtypeStruct((_NW, _L), jnp.float32),
    scratch_types=[
        pltpu.VMEM((_RPW,), jnp.int32),
        pltpu.VMEM((_RPW, _TH, _TW), jnp.float32),
        pltpu.VMEM((_L,), jnp.float32),
        pltpu.SemaphoreType.DMA,
    ],
)(_sc_gather_body)


def kernel(predictions, targets):
    sums = _dense_sum(predictions, targets.reshape(_B, 1))
    total, tail = sums[0, 0], sums[0, 1]
    parts = _sc_gather(predictions, targets)
    return -(_INV_EPS * total + _COEF * (jnp.sum(parts) + tail)) / _B


# speedup vs baseline: 2.7091x; 1.0142x over previous
import functools

import jax
import jax.numpy as jnp
from jax import lax
from jax.experimental import pallas as pl
from jax.experimental.pallas import tpu as pltpu
from jax.experimental.pallas import tpu_sc as plsc

_EPS = 0.1
_NC = 100000
_INV_EPS = _EPS / (_NC - 1)
_B = 1024
_COEF = (1.0 - _EPS) - _INV_EPS

_BLK_R = 64


_TAIL = 32
_TAIL_START = _NC - _TAIL


def _sum_body(x_ref, tgt_ref, o_ref):
    @pl.when(pl.program_id(0) == 0)
    def _init():
        o_ref[0, 0] = 0.0
        o_ref[0, 1] = 0.0

    o_ref[0, 0] += jnp.sum(x_ref[...])
    xtail = x_ref[:, _TAIL_START:]
    col = jax.lax.broadcasted_iota(jnp.int32, (_BLK_R, _TAIL), 1) + _TAIL_START
    hit = col == tgt_ref[...]
    o_ref[0, 1] += jnp.sum(jnp.where(hit, xtail, 0.0))


def _dense_sum(x2d, tgt2d):
    return pl.pallas_call(
        _sum_body,
        grid=(_B // _BLK_R,),
        in_specs=[
            pl.BlockSpec((_BLK_R, _NC), lambda i: (i, 0)),
            pl.BlockSpec((_BLK_R, 1), lambda i: (i, 0)),
        ],
        out_specs=pl.BlockSpec((1, 2), lambda i: (0, 0), memory_space=pltpu.SMEM),
        out_shape=jax.ShapeDtypeStruct((1, 2), jnp.float32),
    )(x2d, tgt2d)


_info = plsc.get_sparse_core_info()
_NCORES = _info.num_cores
_NSUB = _info.num_subcores
_NW = _NCORES * _NSUB
_RPW = _B // _NW
_L = 16


_TW = 128
_TH = 8
_MAXC0 = _NC - _TAIL - _TW


def _sc_gather_body(pred_hbm, tgt_hbm, out_hbm, tgt_v, win_v, part_v, sem):
    wid = lax.axis_index("s") * _NCORES + lax.axis_index("c")
    base = wid * _RPW
    pltpu.sync_copy(tgt_hbm.at[pl.ds(base, _RPW)], tgt_v)
    iota = lax.iota(jnp.int32, _L)
    copies = []
    scalars = []
    tvecs = [tgt_v[pl.ds(c * _L, _L)] for c in range(_RPW // _L)]
    for r in range(_RPW):
        t = tvecs[r // _L][r % _L]
        c0 = pl.multiple_of(
            jnp.minimum(jnp.bitwise_and(t, -_TW), _MAXC0), _TW
        )
        tile_row = pl.multiple_of(base + (r // _TH) * _TH, _TH)
        copies.append(
            pltpu.async_copy(
                pred_hbm.at[pl.ds(tile_row, _TH), pl.ds(c0, _TW)],
                win_v.at[r],
                sem,
            )
        )
        scalars.append((t, c0))
    for cp in copies:
        cp.wait()
    acc = jnp.zeros((_L,), jnp.float32)
    for r in range(_RPW):
        t, c0 = scalars[r]
        lane = jnp.minimum(t - c0, _TW - 1)
        chunk = jnp.bitwise_and(lane, -_L)
        v16 = win_v[r, r % _TH, pl.ds(chunk, _L)]
        hit = jnp.where(iota == lane - chunk, v16, 0.0)
        acc = acc + hit * jnp.where(t < _TAIL_START, 1.0, 0.0)
    part_v[...] = acc
    pltpu.sync_copy(part_v, out_hbm.at[wid])


_sc_gather = functools.partial(
    pl.kernel,
    mesh=plsc.VectorSubcoreMesh(core_axis_name="c", subcore_axis_name="s"),
    out_type=jax.ShapeDtypeStruct((_NW, _L), jnp.float32),
    scratch_types=[
        pltpu.VMEM((_RPW,), jnp.int32),
        pltpu.VMEM((_RPW, _TH, _TW), jnp.float32),
        pltpu.VMEM((_L,), jnp.float32),
        pltpu.SemaphoreType.DMA,
    ],
)(_sc_gather_body)


def kernel(predictions, targets):
    sums = _dense_sum(predictions, targets.reshape(_B, 1))
    total, tail = sums[0, 0], sums[0, 1]
    parts = _sc_gather(predictions, targets)
    return -(_INV_EPS * total + _COEF * (jnp.sum(parts) + tail)) / _B


# speedup vs baseline: 2.7341x; 1.0092x over previous
import functools

import jax
import jax.numpy as jnp
from jax import lax
from jax.experimental import pallas as pl
from jax.experimental.pallas import tpu as pltpu
from jax.experimental.pallas import tpu_sc as plsc

_EPS = 0.1
_NC = 100000
_INV_EPS = _EPS / (_NC - 1)
_B = 1024
_COEF = (1.0 - _EPS) - _INV_EPS

_NSTREAM = 4
_BLK_R = 16


_TAIL = 32
_TAIL_START = _NC - _TAIL


def _sum_body(*refs):
    x_refs = refs[:_NSTREAM]
    tgt_refs = refs[_NSTREAM : 2 * _NSTREAM]
    o_ref = refs[2 * _NSTREAM]

    @pl.when(pl.program_id(0) == 0)
    def _init():
        o_ref[0, 0] = 0.0
        o_ref[0, 1] = 0.0

    s = jnp.sum(x_refs[0][...])
    for k in range(1, _NSTREAM):
        s += jnp.sum(x_refs[k][...])
    o_ref[0, 0] += s
    col = jax.lax.broadcasted_iota(jnp.int32, (_BLK_R, _TAIL), 1) + _TAIL_START
    tail = jnp.zeros((), jnp.float32)
    for k in range(_NSTREAM):
        xtail = x_refs[k][:, _TAIL_START:]
        hit = col == tgt_refs[k][...]
        tail += jnp.sum(jnp.where(hit, xtail, 0.0))
    o_ref[0, 1] += tail


def _dense_sum(x2d, tgt2d):
    x_specs = [
        pl.BlockSpec((_BLK_R, _NC), lambda i, k=k: (i * _NSTREAM + k, 0))
        for k in range(_NSTREAM)
    ]
    t_specs = [
        pl.BlockSpec((_BLK_R, 1), lambda i, k=k: (i * _NSTREAM + k, 0))
        for k in range(_NSTREAM)
    ]
    return pl.pallas_call(
        _sum_body,
        grid=(_B // (_BLK_R * _NSTREAM),),
        in_specs=x_specs + t_specs,
        out_specs=pl.BlockSpec((1, 2), lambda i: (0, 0), memory_space=pltpu.SMEM),
        out_shape=jax.ShapeDtypeStruct((1, 2), jnp.float32),
    )(*([x2d] * _NSTREAM + [tgt2d] * _NSTREAM))


_info = plsc.get_sparse_core_info()
_NCORES = _info.num_cores
_NSUB = _info.num_subcores
_NW = _NCORES * _NSUB
_RPW = _B // _NW
_L = 16


_TW = 128
_TH = 8
_MAXC0 = _NC - _TAIL - _TW


def _sc_gather_body(pred_hbm, tgt_hbm, out_hbm, tgt_v, win_v, part_v, sem):
    wid = lax.axis_index("s") * _NCORES + lax.axis_index("c")
    base = wid * _RPW
    pltpu.sync_copy(tgt_hbm.at[pl.ds(base, _RPW)], tgt_v)
    iota = lax.iota(jnp.int32, _L)
    copies = []
    scalars = []
    tvecs = [tgt_v[pl.ds(c * _L, _L)] for c in range(_RPW // _L)]
    for r in range(_RPW):
        t = tvecs[r // _L][r % _L]
        c0 = pl.multiple_of(
            jnp.minimum(jnp.bitwise_and(t, -_TW), _MAXC0), _TW
        )
        tile_row = pl.multiple_of(base + (r // _TH) * _TH, _TH)
        copies.append(
            pltpu.async_copy(
                pred_hbm.at[pl.ds(tile_row, _TH), pl.ds(c0, _TW)],
                win_v.at[r],
                sem,
            )
        )
        scalars.append((t, c0))
    for cp in copies:
        cp.wait()
    acc = jnp.zeros((_L,), jnp.float32)
    for r in range(_RPW):
        t, c0 = scalars[r]
        lane = jnp.minimum(t - c0, _TW - 1)
        chunk = jnp.bitwise_and(lane, -_L)
        v16 = win_v[r, r % _TH, pl.ds(chunk, _L)]
        hit = jnp.where(iota == lane - chunk, v16, 0.0)
        acc = acc + hit * jnp.where(t < _TAIL_START, 1.0, 0.0)
    part_v[...] = acc
    pltpu.sync_copy(part_v, out_hbm.at[wid])


_sc_gather = functools.partial(
    pl.kernel,
    mesh=plsc.VectorSubcoreMesh(core_axis_name="c", subcore_axis_name="s"),
    out_type=jax.ShapeDtypeStruct((_NW, _L), jnp.float32),
    scratch_types=[
        pltpu.VMEM((_RPW,), jnp.int32),
        pltpu.VMEM((_RPW, _TH, _TW), jnp.float32),
        pltpu.VMEM((_L,), jnp.float32),
        pltpu.SemaphoreType.DMA,
    ],
)(_sc_gather_body)


def kernel(predictions, targets):
    sums = _dense_sum(predictions, targets.reshape(_B, 1))
    total, tail = sums[0, 0], sums[0, 1]
    parts = _sc_gather(predictions, targets)
    return -(_INV_EPS * total + _COEF * (jnp.sum(parts) + tail)) / _B
